# Initial kernel scaffold; baseline (speedup 1.0000x reference)
import functools

import jax
import jax.numpy as jnp
from jax import lax
from jax.experimental import pallas as pl
from jax.experimental.pallas import tpu as pltpu
from jax.experimental.pallas import tpu_sc as plsc

NC = 2
NS = 16
NW = NC * NS
LANES = 16
SUB = 128
CHUNK = 1024


def _mesh():
    return plsc.VectorSubcoreMesh(core_axis_name="c", subcore_axis_name="s")


def _sc_deg(dst2d, ew_flat, n_pad, n_chunks):
    slice_n = n_pad // NS
    rpw = n_chunks * (CHUNK // SUB)

    @functools.partial(
        pl.kernel,
        out_type=jax.ShapeDtypeStruct((NC, n_pad), jnp.float32),
        mesh=_mesh(),
        scratch_types=[
            pltpu.VMEM((CHUNK // SUB, SUB), jnp.int32),
            pltpu.VMEM((CHUNK,), jnp.float32),
            pltpu.VMEM((slice_n,), jnp.float32),
            pltpu.VMEM_SHARED((n_pad,), jnp.float32),
            pltpu.SemaphoreType.DMA,
        ],
    )
    def k(dst_hbm, ew_hbm, out_hbm, dstv, eww, zbuf, acc, sem):
        c = lax.axis_index("c")
        s = lax.axis_index("s")
        wid = c * NS + s

        @pl.loop(0, slice_n // LANES)
        def _(i):
            zbuf[pl.ds(i * LANES, LANES)] = jnp.zeros((LANES,), jnp.float32)

        pltpu.sync_copy(zbuf, acc.at[pl.ds(s * slice_n, slice_n)])
        plsc.subcore_barrier()

        @pl.loop(0, n_chunks)
        def _(ch):
            row0 = wid * rpw + ch * (CHUNK // SUB)
            base = wid * (n_chunks * CHUNK) + ch * CHUNK
            pltpu.sync_copy(dst_hbm.at[pl.ds(row0, CHUNK // SUB)], dstv)
            pltpu.sync_copy(ew_hbm.at[pl.ds(base, CHUNK)], eww)
            cps = [
                pltpu.async_copy(
                    eww.at[pl.ds(j * SUB, SUB)], acc.at[dstv.at[j]], sem,
                    add=True)
                for j in range(CHUNK // SUB)
            ]
            for cp in cps:
                cp.wait()

        plsc.subcore_barrier()
        pltpu.sync_copy(acc.at[pl.ds(s * slice_n, slice_n)],
                        out_hbm.at[c].at[pl.ds(s * slice_n, slice_n)])

    return k(dst2d, ew_flat)


def _sc_agg(y, src2d, dst2d, ew_flat, h, n_pad, n_chunks):
    slice_n = n_pad // NS
    rpw = n_chunks * (CHUNK // SUB)

    @functools.partial(
        pl.kernel,
        out_type=jax.ShapeDtypeStruct((NC, n_pad, h), jnp.float32),
        mesh=_mesh(),
        scratch_types=[
            pltpu.VMEM((CHUNK // SUB, SUB), jnp.int32),
            pltpu.VMEM((CHUNK // SUB, SUB), jnp.int32),
            pltpu.VMEM((CHUNK,), jnp.float32),
            pltpu.VMEM((CHUNK, h), jnp.float32),
            pltpu.VMEM_SHARED((n_pad, h), jnp.float32),
            pltpu.SemaphoreType.DMA,
            pltpu.SemaphoreType.DMA,
        ],
    )
    def k(y_hbm, src_hbm, dst_hbm, ew_hbm, out_hbm,
          srcv, dstv, eww, rows, acc, gsem, ssem):
        c = lax.axis_index("c")
        s = lax.axis_index("s")
        wid = c * NS + s

        @pl.loop(0, slice_n)
        def _(i):
            for kk in range(h // LANES):
                rows[i, pl.ds(kk * LANES, LANES)] = jnp.zeros(
                    (LANES,), jnp.float32)

        pltpu.sync_copy(rows.at[pl.ds(0, slice_n)],
                        acc.at[pl.ds(s * slice_n, slice_n)])
        plsc.subcore_barrier()

        @pl.loop(0, n_chunks)
        def _(ch):
            row0 = wid * rpw + ch * (CHUNK // SUB)
            base = wid * (n_chunks * CHUNK) + ch * CHUNK
            pltpu.sync_copy(src_hbm.at[pl.ds(row0, CHUNK // SUB)], srcv)
            pltpu.sync_copy(dst_hbm.at[pl.ds(row0, CHUNK // SUB)], dstv)
            pltpu.sync_copy(ew_hbm.at[pl.ds(base, CHUNK)], eww)
            gcp = [
                pltpu.async_copy(y_hbm.at[srcv.at[j]],
                                 rows.at[pl.ds(j * SUB, SUB)], gsem)
                for j in range(CHUNK // SUB)
            ]
            for cp in gcp:
                cp.wait()

            @pl.loop(0, CHUNK // LANES)
            def _(g):
                wreg = eww[pl.ds(g * LANES, LANES)]
                for j in range(LANES):
                    e = g * LANES + j
                    wj = wreg[j]
                    for kk in range(h // LANES):
                        sl = pl.ds(kk * LANES, LANES)
                        rows[e, sl] = rows[e, sl] * wj

            scp = [
                pltpu.async_copy(rows.at[pl.ds(j * SUB, SUB)],
                                 acc.at[dstv.at[j]], ssem, add=True)
                for j in range(CHUNK // SUB)
            ]
            for cp in scp:
                cp.wait()

        plsc.subcore_barrier()
        pltpu.sync_copy(acc.at[pl.ds(s * slice_n, slice_n)],
                        out_hbm.at[c].at[pl.ds(s * slice_n, slice_n)])

    return k(y, src2d, dst2d, ew_flat)


def _mm_kernel(x_ref, w_ref, o_ref):
    o_ref[...] = jnp.dot(x_ref[...], w_ref[...])


def _tc_matmul(x, w):
    return pl.pallas_call(
        _mm_kernel,
        out_shape=jax.ShapeDtypeStruct((x.shape[0], w.shape[1]), jnp.float32),
    )(x, w)


def _dinv_of(degp_ref, n):
    deg = degp_ref[0] + degp_ref[1] + 1.0
    return jnp.where(deg > 0, lax.rsqrt(deg), 0.0)[:n]


def _scale_kernel(n, degp_ref, xw_ref, y_ref):
    y_ref[...] = xw_ref[...] * _dinv_of(degp_ref, n)


def _tc_scale(degp_col, xw):
    n = xw.shape[0]
    return pl.pallas_call(
        functools.partial(_scale_kernel, n),
        out_shape=jax.ShapeDtypeStruct(xw.shape, jnp.float32),
    )(degp_col, xw)


def _mid_kernel(n, degp_ref, s1_ref, xw_ref, b1_ref, w2_ref,
                hw2_ref, y2_ref):
    dinv = _dinv_of(degp_ref, n)
    s1 = s1_ref[0, :n] + s1_ref[1, :n]
    h = jnp.maximum(dinv * s1 + dinv * dinv * xw_ref[...] + b1_ref[...], 0.0)
    hw2 = jnp.dot(h, w2_ref[...])
    hw2_ref[...] = hw2
    y2_ref[...] = hw2 * dinv


def _tc_mid(degp_col, s1, xw, b1_row, w2):
    n = xw.shape[0]
    h2 = w2.shape[1]
    return pl.pallas_call(
        functools.partial(_mid_kernel, n),
        out_shape=(
            jax.ShapeDtypeStruct((n, h2), jnp.float32),
            jax.ShapeDtypeStruct((n, h2), jnp.float32),
        ),
    )(degp_col, s1, xw, b1_row, w2)


def _final_kernel(n, degp_ref, s2_ref, hw2_ref, b2_ref, o_ref):
    dinv = _dinv_of(degp_ref, n)
    s2 = s2_ref[0, :n] + s2_ref[1, :n]
    o_ref[...] = dinv * s2 + dinv * dinv * hw2_ref[...] + b2_ref[...]


def _tc_final(degp_col, s2, hw2, b2_row):
    return pl.pallas_call(
        functools.partial(_final_kernel, hw2.shape[0]),
        out_shape=jax.ShapeDtypeStruct(hw2.shape, jnp.float32),
    )(degp_col, s2, hw2, b2_row)


@jax.jit
def kernel(x, edge_index, edge_weight, W1, b1, W2, b2):
    n, _ = x.shape
    e = edge_weight.shape[0]

    n_chunks = -(-e // (NW * CHUNK))
    e_pad = NW * CHUNK * n_chunks
    pad = e_pad - e
    n_pad = -(-n // (NS * 8)) * (NS * 8)

    fill = jnp.arange(pad, dtype=jnp.int32) % n
    src_p = jnp.concatenate([edge_index[0], fill]).reshape(e_pad // SUB, SUB)
    dst_p = jnp.concatenate([edge_index[1], fill]).reshape(e_pad // SUB, SUB)
    ew_p = jnp.concatenate(
        [edge_weight, jnp.zeros((pad,), jnp.float32)])

    xw = _tc_matmul(x, W1)
    degp = _sc_deg(dst_p, ew_p, n_pad, n_chunks)
    degp_col = degp.reshape(NC, n_pad, 1)

    y1 = _tc_scale(degp_col, xw)
    s1 = _sc_agg(y1, src_p, dst_p, ew_p, W1.shape[1], n_pad, n_chunks)
    hw2, y2 = _tc_mid(degp_col, s1, xw, b1.reshape(1, -1), W2)
    s2 = _sc_agg(y2, src_p, dst_p, ew_p, W2.shape[1], n_pad, n_chunks)
    return _tc_final(degp_col, s2, hw2, b2.reshape(1, -1))


# speedup vs baseline: 37.4669x; 37.4669x over previous
import functools

import jax
import jax.numpy as jnp
from jax import lax
from jax.experimental import pallas as pl
from jax.experimental.pallas import tpu as pltpu
from jax.experimental.pallas import tpu_sc as plsc

NC = 2
NS = 16
NW = NC * NS
LANES = 16
SUB = 128
CHUNK = 1024


def _mesh():
    return plsc.VectorSubcoreMesh(core_axis_name="c", subcore_axis_name="s")


def _sc_deg(dst2d, ew_flat, n_pad, n_chunks):
    slice_n = n_pad // NS
    rpw = n_chunks * (CHUNK // SUB)

    @functools.partial(
        pl.kernel,
        out_type=jax.ShapeDtypeStruct((NC, n_pad), jnp.float32),
        mesh=_mesh(),
        scratch_types=[
            pltpu.VMEM((CHUNK // SUB, SUB), jnp.int32),
            pltpu.VMEM((CHUNK,), jnp.float32),
            pltpu.VMEM((slice_n,), jnp.float32),
            pltpu.VMEM_SHARED((n_pad,), jnp.float32),
            pltpu.SemaphoreType.DMA,
        ],
    )
    def k(dst_hbm, ew_hbm, out_hbm, dstv, eww, zbuf, acc, sem):
        c = lax.axis_index("c")
        s = lax.axis_index("s")
        wid = c * NS + s

        @pl.loop(0, slice_n // LANES)
        def _(i):
            zbuf[pl.ds(i * LANES, LANES)] = jnp.zeros((LANES,), jnp.float32)

        pltpu.sync_copy(zbuf, acc.at[pl.ds(s * slice_n, slice_n)])
        plsc.subcore_barrier()

        @pl.loop(0, n_chunks)
        def _(ch):
            row0 = wid * rpw + ch * (CHUNK // SUB)
            base = wid * (n_chunks * CHUNK) + ch * CHUNK
            pltpu.sync_copy(dst_hbm.at[pl.ds(row0, CHUNK // SUB)], dstv)
            pltpu.sync_copy(ew_hbm.at[pl.ds(base, CHUNK)], eww)
            cps = [
                pltpu.async_copy(
                    eww.at[pl.ds(j * SUB, SUB)], acc.at[dstv.at[j]], sem,
                    add=True)
                for j in range(CHUNK // SUB)
            ]
            for cp in cps:
                cp.wait()

        plsc.subcore_barrier()
        pltpu.sync_copy(acc.at[pl.ds(s * slice_n, slice_n)],
                        out_hbm.at[c].at[pl.ds(s * slice_n, slice_n)])

    return k(dst2d, ew_flat)


def _sc_agg(y, src2d, dst2d, ew_flat, h, n_pad, n_chunks):
    slice_n = n_pad // NS
    rpw = n_chunks * (CHUNK // SUB)

    @functools.partial(
        pl.kernel,
        out_type=jax.ShapeDtypeStruct((NC, n_pad, h), jnp.float32),
        mesh=_mesh(),
        scratch_types=[
            pltpu.VMEM((CHUNK // SUB, SUB), jnp.int32),
            pltpu.VMEM((CHUNK // SUB, SUB), jnp.int32),
            pltpu.VMEM((CHUNK,), jnp.float32),
            pltpu.VMEM((CHUNK, h), jnp.float32),
            pltpu.VMEM_SHARED((n_pad, h), jnp.float32),
            pltpu.SemaphoreType.DMA,
            pltpu.SemaphoreType.DMA,
        ],
        compiler_params=pltpu.CompilerParams(use_tc_tiling_on_sc=False),
    )
    def k(y_hbm, src_hbm, dst_hbm, ew_hbm, out_hbm,
          srcv, dstv, eww, rows, acc, gsem, ssem):
        c = lax.axis_index("c")
        s = lax.axis_index("s")
        wid = c * NS + s

        @pl.loop(0, slice_n)
        def _(i):
            for kk in range(h // LANES):
                rows[i, pl.ds(kk * LANES, LANES)] = jnp.zeros(
                    (LANES,), jnp.float32)

        pltpu.sync_copy(rows.at[pl.ds(0, slice_n)],
                        acc.at[pl.ds(s * slice_n, slice_n)])
        plsc.subcore_barrier()

        @pl.loop(0, n_chunks)
        def _(ch):
            row0 = wid * rpw + ch * (CHUNK // SUB)
            base = wid * (n_chunks * CHUNK) + ch * CHUNK
            pltpu.sync_copy(src_hbm.at[pl.ds(row0, CHUNK // SUB)], srcv)
            pltpu.sync_copy(dst_hbm.at[pl.ds(row0, CHUNK // SUB)], dstv)
            pltpu.sync_copy(ew_hbm.at[pl.ds(base, CHUNK)], eww)
            gcp = [
                pltpu.async_copy(y_hbm.at[srcv.at[j]],
                                 rows.at[pl.ds(j * SUB, SUB)], gsem)
                for j in range(CHUNK // SUB)
            ]
            for cp in gcp:
                cp.wait()

            @pl.loop(0, CHUNK // LANES)
            def _(g):
                wreg = eww[pl.ds(g * LANES, LANES)]
                for j in range(LANES):
                    e = g * LANES + j
                    wj = wreg[j]
                    for kk in range(h // LANES):
                        sl = pl.ds(kk * LANES, LANES)
                        rows[e, sl] = rows[e, sl] * wj

            scp = [
                pltpu.async_copy(rows.at[pl.ds(j * SUB, SUB)],
                                 acc.at[dstv.at[j]], ssem, add=True)
                for j in range(CHUNK // SUB)
            ]
            for cp in scp:
                cp.wait()

        plsc.subcore_barrier()
        pltpu.sync_copy(acc.at[pl.ds(s * slice_n, slice_n)],
                        out_hbm.at[c].at[pl.ds(s * slice_n, slice_n)])

    return k(y, src2d, dst2d, ew_flat)


def _mm_kernel(x_ref, w_ref, o_ref):
    o_ref[...] = jnp.dot(x_ref[...], w_ref[...])


def _tc_matmul(x, w):
    return pl.pallas_call(
        _mm_kernel,
        out_shape=jax.ShapeDtypeStruct((x.shape[0], w.shape[1]), jnp.float32),
    )(x, w)


def _dinv_of(degp_ref, n):
    deg = degp_ref[0] + degp_ref[1] + 1.0
    return jnp.where(deg > 0, lax.rsqrt(deg), 0.0)[:n]


def _scale_kernel(n, degp_ref, xw_ref, y_ref):
    y_ref[...] = xw_ref[...] * _dinv_of(degp_ref, n)


def _tc_scale(degp_col, xw):
    n = xw.shape[0]
    return pl.pallas_call(
        functools.partial(_scale_kernel, n),
        out_shape=jax.ShapeDtypeStruct(xw.shape, jnp.float32),
    )(degp_col, xw)


def _mid_kernel(n, degp_ref, s1_ref, xw_ref, b1_ref, w2_ref,
                hw2_ref, y2_ref):
    dinv = _dinv_of(degp_ref, n)
    s1 = s1_ref[0, :n] + s1_ref[1, :n]
    h = jnp.maximum(dinv * s1 + dinv * dinv * xw_ref[...] + b1_ref[...], 0.0)
    hw2 = jnp.dot(h, w2_ref[...])
    hw2_ref[...] = hw2
    y2_ref[...] = hw2 * dinv


def _tc_mid(degp_col, s1, xw, b1_row, w2):
    n = xw.shape[0]
    h2 = w2.shape[1]
    return pl.pallas_call(
        functools.partial(_mid_kernel, n),
        out_shape=(
            jax.ShapeDtypeStruct((n, h2), jnp.float32),
            jax.ShapeDtypeStruct((n, h2), jnp.float32),
        ),
    )(degp_col, s1, xw, b1_row, w2)


def _final_kernel(n, degp_ref, s2_ref, hw2_ref, b2_ref, o_ref):
    dinv = _dinv_of(degp_ref, n)
    s2 = s2_ref[0, :n] + s2_ref[1, :n]
    o_ref[...] = dinv * s2 + dinv * dinv * hw2_ref[...] + b2_ref[...]


def _tc_final(degp_col, s2, hw2, b2_row):
    return pl.pallas_call(
        functools.partial(_final_kernel, hw2.shape[0]),
        out_shape=jax.ShapeDtypeStruct(hw2.shape, jnp.float32),
    )(degp_col, s2, hw2, b2_row)


@jax.jit
def kernel(x, edge_index, edge_weight, W1, b1, W2, b2):
    n, _ = x.shape
    e = edge_weight.shape[0]

    n_chunks = -(-e // (NW * CHUNK))
    e_pad = NW * CHUNK * n_chunks
    pad = e_pad - e
    n_pad = -(-n // (NS * SUB)) * (NS * SUB)

    fill = jnp.arange(pad, dtype=jnp.int32) % n
    src_p = jnp.concatenate([edge_index[0], fill]).reshape(e_pad // SUB, SUB)
    dst_p = jnp.concatenate([edge_index[1], fill]).reshape(e_pad // SUB, SUB)
    ew_p = jnp.concatenate(
        [edge_weight, jnp.zeros((pad,), jnp.float32)])

    xw = _tc_matmul(x, W1)
    degp = _sc_deg(dst_p, ew_p, n_pad, n_chunks)
    degp_col = degp.reshape(NC, n_pad, 1)

    y1 = _tc_scale(degp_col, xw)
    s1 = _sc_agg(y1, src_p, dst_p, ew_p, W1.shape[1], n_pad, n_chunks)
    hw2, y2 = _tc_mid(degp_col, s1, xw, b1.reshape(1, -1), W2)
    s2 = _sc_agg(y2, src_p, dst_p, ew_p, W2.shape[1], n_pad, n_chunks)
    return _tc_final(degp_col, s2, hw2, b2.reshape(1, -1))


# speedup vs baseline: 39.1470x; 1.0448x over previous
import functools

import jax
import jax.numpy as jnp
from jax import lax
from jax.experimental import pallas as pl
from jax.experimental.pallas import tpu as pltpu
from jax.experimental.pallas import tpu_sc as plsc

NC = 2
NS = 16
NW = NC * NS
LANES = 16
SUB = 128
CHUNK = 1024


def _mesh():
    return plsc.VectorSubcoreMesh(core_axis_name="c", subcore_axis_name="s")


def _sc_deg(dst2d, ew_flat, n_pad, n_chunks):
    slice_n = n_pad // NS
    rpw = n_chunks * (CHUNK // SUB)

    @functools.partial(
        pl.kernel,
        out_type=jax.ShapeDtypeStruct((NC, n_pad), jnp.float32),
        mesh=_mesh(),
        scratch_types=[
            pltpu.VMEM((CHUNK // SUB, SUB), jnp.int32),
            pltpu.VMEM((CHUNK,), jnp.float32),
            pltpu.VMEM((slice_n,), jnp.float32),
            pltpu.VMEM_SHARED((n_pad,), jnp.float32),
            pltpu.SemaphoreType.DMA,
        ],
    )
    def k(dst_hbm, ew_hbm, out_hbm, dstv, eww, zbuf, acc, sem):
        c = lax.axis_index("c")
        s = lax.axis_index("s")
        wid = c * NS + s

        @pl.loop(0, slice_n // LANES)
        def _(i):
            zbuf[pl.ds(i * LANES, LANES)] = jnp.zeros((LANES,), jnp.float32)

        pltpu.sync_copy(zbuf, acc.at[pl.ds(s * slice_n, slice_n)])
        plsc.subcore_barrier()

        @pl.loop(0, n_chunks)
        def _(ch):
            row0 = wid * rpw + ch * (CHUNK // SUB)
            base = wid * (n_chunks * CHUNK) + ch * CHUNK
            pltpu.sync_copy(dst_hbm.at[pl.ds(row0, CHUNK // SUB)], dstv)
            pltpu.sync_copy(ew_hbm.at[pl.ds(base, CHUNK)], eww)
            cps = [
                pltpu.async_copy(
                    eww.at[pl.ds(j * SUB, SUB)], acc.at[dstv.at[j]], sem,
                    add=True)
                for j in range(CHUNK // SUB)
            ]
            for cp in cps:
                cp.wait()

        plsc.subcore_barrier()
        pltpu.sync_copy(acc.at[pl.ds(s * slice_n, slice_n)],
                        out_hbm.at[c].at[pl.ds(s * slice_n, slice_n)])

    return k(dst2d, ew_flat)


def _sc_agg(y, src2d, dst2d, ew_flat, h, n_pad, n_chunks):
    slice_n = n_pad // NS
    rpw = n_chunks * (CHUNK // SUB)

    @functools.partial(
        pl.kernel,
        out_type=jax.ShapeDtypeStruct((NC, n_pad, h), jnp.float32),
        mesh=_mesh(),
        scratch_types=[
            pltpu.VMEM((CHUNK // SUB, SUB), jnp.int32),
            pltpu.VMEM((CHUNK // SUB, SUB), jnp.int32),
            pltpu.VMEM((CHUNK,), jnp.float32),
            pltpu.VMEM((CHUNK, h), jnp.float32),
            pltpu.VMEM_SHARED((n_pad, h), jnp.float32),
            pltpu.SemaphoreType.DMA,
            pltpu.SemaphoreType.DMA,
        ],
        compiler_params=pltpu.CompilerParams(use_tc_tiling_on_sc=False),
    )
    def k(y_hbm, src_hbm, dst_hbm, ew_hbm, out_hbm,
          srcv, dstv, eww, rows, acc, gsem, ssem):
        c = lax.axis_index("c")
        s = lax.axis_index("s")
        wid = c * NS + s

        @pl.loop(0, slice_n)
        def _(i):
            for kk in range(h // LANES):
                rows[i, pl.ds(kk * LANES, LANES)] = jnp.zeros(
                    (LANES,), jnp.float32)

        pltpu.sync_copy(rows.at[pl.ds(0, slice_n)],
                        acc.at[pl.ds(s * slice_n, slice_n)])
        plsc.subcore_barrier()

        @pl.loop(0, n_chunks)
        def _(ch):
            row0 = wid * rpw + ch * (CHUNK // SUB)
            base = wid * (n_chunks * CHUNK) + ch * CHUNK
            pltpu.sync_copy(src_hbm.at[pl.ds(row0, CHUNK // SUB)], srcv)
            pltpu.sync_copy(dst_hbm.at[pl.ds(row0, CHUNK // SUB)], dstv)
            pltpu.sync_copy(ew_hbm.at[pl.ds(base, CHUNK)], eww)
            gcp = [
                pltpu.async_copy(y_hbm.at[srcv.at[j]],
                                 rows.at[pl.ds(j * SUB, SUB)], gsem)
                for j in range(CHUNK // SUB)
            ]
            for cp in gcp:
                cp.wait()

            @pl.loop(0, CHUNK // LANES)
            def _(g):
                wreg = eww[pl.ds(g * LANES, LANES)]
                for j in range(LANES):
                    e = g * LANES + j
                    wj = wreg[j]
                    for kk in range(h // LANES):
                        sl = pl.ds(kk * LANES, LANES)
                        rows[e, sl] = rows[e, sl] * wj

            scp = [
                pltpu.async_copy(rows.at[pl.ds(j * SUB, SUB)],
                                 acc.at[dstv.at[j]], ssem, add=True)
                for j in range(CHUNK // SUB)
            ]
            for cp in scp:
                cp.wait()

        plsc.subcore_barrier()
        pltpu.sync_copy(acc.at[pl.ds(s * slice_n, slice_n)],
                        out_hbm.at[c].at[pl.ds(s * slice_n, slice_n)])

    return k(y, src2d, dst2d, ew_flat)


def _mm_kernel(x_ref, w_ref, o_ref):
    o_ref[...] = jnp.dot(x_ref[...], w_ref[...])


def _tc_matmul(x, w):
    return pl.pallas_call(
        _mm_kernel,
        out_shape=jax.ShapeDtypeStruct((x.shape[0], w.shape[1]), jnp.float32),
    )(x, w)


def _scale_kernel(n, degp_ref, xw_ref, y_ref, dinv_ref):
    deg = degp_ref[0] + degp_ref[1] + 1.0
    dinv = jnp.where(deg > 0, lax.rsqrt(deg), 0.0)
    dinv_col = dinv.reshape(deg.shape[0], 1)[:n]
    y_ref[...] = xw_ref[...] * dinv_col
    dinv_ref[...] = dinv_col


def _tc_scale(degp, xw):
    n = xw.shape[0]
    return pl.pallas_call(
        functools.partial(_scale_kernel, n),
        out_shape=(
            jax.ShapeDtypeStruct(xw.shape, jnp.float32),
            jax.ShapeDtypeStruct((n, 1), jnp.float32),
        ),
    )(degp, xw)


def _mid_kernel(n, dinv_ref, s1_ref, xw_ref, b1_ref, w2_ref,
                hw2_ref, y2_ref):
    dinv = dinv_ref[...]
    s1 = s1_ref[0, :n] + s1_ref[1, :n]
    h = jnp.maximum(dinv * s1 + dinv * dinv * xw_ref[...] + b1_ref[...], 0.0)
    hw2 = jnp.dot(h, w2_ref[...])
    hw2_ref[...] = hw2
    y2_ref[...] = hw2 * dinv


def _tc_mid(dinv_col, s1, xw, b1_row, w2):
    n = xw.shape[0]
    h2 = w2.shape[1]
    return pl.pallas_call(
        functools.partial(_mid_kernel, n),
        out_shape=(
            jax.ShapeDtypeStruct((n, h2), jnp.float32),
            jax.ShapeDtypeStruct((n, h2), jnp.float32),
        ),
    )(dinv_col, s1, xw, b1_row, w2)


def _final_kernel(n, dinv_ref, s2_ref, hw2_ref, b2_ref, o_ref):
    dinv = dinv_ref[...]
    s2 = s2_ref[0, :n] + s2_ref[1, :n]
    o_ref[...] = dinv * s2 + dinv * dinv * hw2_ref[...] + b2_ref[...]


def _tc_final(dinv_col, s2, hw2, b2_row):
    return pl.pallas_call(
        functools.partial(_final_kernel, hw2.shape[0]),
        out_shape=jax.ShapeDtypeStruct(hw2.shape, jnp.float32),
    )(dinv_col, s2, hw2, b2_row)


@jax.jit
def kernel(x, edge_index, edge_weight, W1, b1, W2, b2):
    n, _ = x.shape
    e = edge_weight.shape[0]

    n_chunks = -(-e // (NW * CHUNK))
    e_pad = NW * CHUNK * n_chunks
    pad = e_pad - e
    n_pad = -(-n // (NS * SUB)) * (NS * SUB)

    fill = jnp.arange(pad, dtype=jnp.int32) % n
    src_p = jnp.concatenate([edge_index[0], fill]).reshape(e_pad // SUB, SUB)
    dst_p = jnp.concatenate([edge_index[1], fill]).reshape(e_pad // SUB, SUB)
    ew_p = jnp.concatenate(
        [edge_weight, jnp.zeros((pad,), jnp.float32)])

    xw = _tc_matmul(x, W1)
    degp = _sc_deg(dst_p, ew_p, n_pad, n_chunks)

    y1, dinv_col = _tc_scale(degp, xw)
    s1 = _sc_agg(y1, src_p, dst_p, ew_p, W1.shape[1], n_pad, n_chunks)
    hw2, y2 = _tc_mid(dinv_col, s1, xw, b1.reshape(1, -1), W2)
    s2 = _sc_agg(y2, src_p, dst_p, ew_p, W2.shape[1], n_pad, n_chunks)
    return _tc_final(dinv_col, s2, hw2, b2.reshape(1, -1))


# speedup vs baseline: 45.5635x; 1.1639x over previous
import functools

import jax
import jax.numpy as jnp
from jax import lax
from jax.experimental import pallas as pl
from jax.experimental.pallas import tpu as pltpu
from jax.experimental.pallas import tpu_sc as plsc

NC = 2
NS = 16
NW = NC * NS
LANES = 16
SUB = 128
CHUNK = 1024


def _mesh():
    return plsc.VectorSubcoreMesh(core_axis_name="c", subcore_axis_name="s")


def _sc_deg(dst2d, ew_flat, n_pad, n_chunks):
    slice_n = n_pad // NS
    rpw = n_chunks * (CHUNK // SUB)

    @functools.partial(
        pl.kernel,
        out_type=jax.ShapeDtypeStruct((NC, n_pad), jnp.float32),
        mesh=_mesh(),
        scratch_types=[
            pltpu.VMEM((CHUNK // SUB, SUB), jnp.int32),
            pltpu.VMEM((CHUNK,), jnp.float32),
            pltpu.VMEM((slice_n,), jnp.float32),
            pltpu.VMEM_SHARED((n_pad,), jnp.float32),
            pltpu.SemaphoreType.DMA,
        ],
    )
    def k(dst_hbm, ew_hbm, out_hbm, dstv, eww, zbuf, acc, sem):
        c = lax.axis_index("c")
        s = lax.axis_index("s")
        wid = c * NS + s

        @pl.loop(0, slice_n // LANES)
        def _(i):
            zbuf[pl.ds(i * LANES, LANES)] = jnp.zeros((LANES,), jnp.float32)

        pltpu.sync_copy(zbuf, acc.at[pl.ds(s * slice_n, slice_n)])
        plsc.subcore_barrier()

        @pl.loop(0, n_chunks)
        def _(ch):
            row0 = wid * rpw + ch * (CHUNK // SUB)
            base = wid * (n_chunks * CHUNK) + ch * CHUNK
            pltpu.sync_copy(dst_hbm.at[pl.ds(row0, CHUNK // SUB)], dstv)
            pltpu.sync_copy(ew_hbm.at[pl.ds(base, CHUNK)], eww)
            cps = [
                pltpu.async_copy(
                    eww.at[pl.ds(j * SUB, SUB)], acc.at[dstv.at[j]], sem,
                    add=True)
                for j in range(CHUNK // SUB)
            ]
            for cp in cps:
                cp.wait()

        plsc.subcore_barrier()
        pltpu.sync_copy(acc.at[pl.ds(s * slice_n, slice_n)],
                        out_hbm.at[c].at[pl.ds(s * slice_n, slice_n)])

    return k(dst2d, ew_flat)


def _sc_agg(y, src2d, dst2d, ew_flat, h, n_pad, n_chunks):
    slice_n = n_pad // NS
    rpw = n_chunks * (CHUNK // SUB)

    nsub = CHUNK // SUB
    assert n_chunks % 2 == 0

    @functools.partial(
        pl.kernel,
        out_type=jax.ShapeDtypeStruct((NC, n_pad, h), jnp.float32),
        mesh=_mesh(),
        scratch_types=[
            pltpu.VMEM((2, nsub, SUB), jnp.int32),
            pltpu.VMEM((2, nsub, SUB), jnp.int32),
            pltpu.VMEM((2, CHUNK), jnp.float32),
            pltpu.VMEM((2, CHUNK, h), jnp.float32),
            pltpu.VMEM_SHARED((n_pad, h), jnp.float32),
            pltpu.SemaphoreType.DMA,
            pltpu.SemaphoreType.DMA,
            pltpu.SemaphoreType.DMA,
            pltpu.SemaphoreType.DMA,
        ],
        compiler_params=pltpu.CompilerParams(use_tc_tiling_on_sc=False),
    )
    def k(y_hbm, src_hbm, dst_hbm, ew_hbm, out_hbm,
          srcv, dstv, eww, rows, acc, gsem0, gsem1, ssem0, ssem1):
        c = lax.axis_index("c")
        s = lax.axis_index("s")
        wid = c * NS + s
        gsem = (gsem0, gsem1)
        ssem = (ssem0, ssem1)

        @pl.loop(0, slice_n)
        def _(i):
            for kk in range(h // LANES):
                rows[0, i, pl.ds(kk * LANES, LANES)] = jnp.zeros(
                    (LANES,), jnp.float32)

        pltpu.sync_copy(rows.at[0, pl.ds(0, slice_n)],
                        acc.at[pl.ds(s * slice_n, slice_n)])
        plsc.subcore_barrier()

        def load_idx(ch, b):
            row0 = wid * rpw + ch * nsub
            base = wid * (n_chunks * CHUNK) + ch * CHUNK
            pltpu.sync_copy(src_hbm.at[pl.ds(row0, nsub)], srcv.at[b])
            pltpu.sync_copy(dst_hbm.at[pl.ds(row0, nsub)], dstv.at[b])
            pltpu.sync_copy(ew_hbm.at[pl.ds(base, CHUNK)], eww.at[b])

        def fire_gather(b):
            for j in range(nsub):
                pltpu.async_copy(y_hbm.at[srcv.at[b].at[j]],
                                 rows.at[b].at[pl.ds(j * SUB, SUB)], gsem[b])

        def wait_gather(b):
            for j in range(nsub):
                pltpu.make_async_copy(
                    y_hbm.at[srcv.at[b].at[j]],
                    rows.at[b].at[pl.ds(j * SUB, SUB)], gsem[b]).wait()

        def fire_scatter(b):
            for j in range(nsub):
                pltpu.async_copy(rows.at[b].at[pl.ds(j * SUB, SUB)],
                                 acc.at[dstv.at[b].at[j]], ssem[b], add=True)

        def wait_scatter(b):
            for j in range(nsub):
                pltpu.make_async_copy(
                    rows.at[b].at[pl.ds(j * SUB, SUB)],
                    acc.at[dstv.at[b].at[j]], ssem[b]).wait()

        def compute(b):
            @pl.loop(0, CHUNK // LANES)
            def _(g):
                wreg = eww.at[b][pl.ds(g * LANES, LANES)]
                for j in range(LANES):
                    e = g * LANES + j
                    wj = wreg[j]
                    for kk in range(h // LANES):
                        sl = pl.ds(kk * LANES, LANES)
                        rows[b, e, sl] = rows[b, e, sl] * wj

        load_idx(0, 0)
        fire_gather(0)

        @pl.loop(0, n_chunks // 2)
        def _(i):
            c0 = 2 * i

            @pl.when(i > 0)
            def _():
                wait_scatter(1)
            load_idx(c0 + 1, 1)
            fire_gather(1)

            wait_gather(0)
            compute(0)
            fire_scatter(0)

            wait_scatter(0)

            @pl.when(c0 + 2 < n_chunks)
            def _():
                load_idx(c0 + 2, 0)
                fire_gather(0)

            wait_gather(1)
            compute(1)
            fire_scatter(1)

        wait_scatter(1)
        plsc.subcore_barrier()
        pltpu.sync_copy(acc.at[pl.ds(s * slice_n, slice_n)],
                        out_hbm.at[c].at[pl.ds(s * slice_n, slice_n)])

    return k(y, src2d, dst2d, ew_flat)


def _mm_kernel(x_ref, w_ref, o_ref):
    o_ref[...] = jnp.dot(x_ref[...], w_ref[...])


def _tc_matmul(x, w):
    return pl.pallas_call(
        _mm_kernel,
        out_shape=jax.ShapeDtypeStruct((x.shape[0], w.shape[1]), jnp.float32),
    )(x, w)


def _scale_kernel(n, degp_ref, xw_ref, y_ref, dinv_ref):
    deg = degp_ref[0] + degp_ref[1] + 1.0
    dinv = jnp.where(deg > 0, lax.rsqrt(deg), 0.0)
    dinv_col = dinv.reshape(deg.shape[0], 1)[:n]
    y_ref[...] = xw_ref[...] * dinv_col
    dinv_ref[...] = dinv_col


def _tc_scale(degp, xw):
    n = xw.shape[0]
    return pl.pallas_call(
        functools.partial(_scale_kernel, n),
        out_shape=(
            jax.ShapeDtypeStruct(xw.shape, jnp.float32),
            jax.ShapeDtypeStruct((n, 1), jnp.float32),
        ),
    )(degp, xw)


def _mid_kernel(n, dinv_ref, s1_ref, xw_ref, b1_ref, w2_ref,
                hw2_ref, y2_ref):
    dinv = dinv_ref[...]
    s1 = s1_ref[0, :n] + s1_ref[1, :n]
    h = jnp.maximum(dinv * s1 + dinv * dinv * xw_ref[...] + b1_ref[...], 0.0)
    hw2 = jnp.dot(h, w2_ref[...])
    hw2_ref[...] = hw2
    y2_ref[...] = hw2 * dinv


def _tc_mid(dinv_col, s1, xw, b1_row, w2):
    n = xw.shape[0]
    h2 = w2.shape[1]
    return pl.pallas_call(
        functools.partial(_mid_kernel, n),
        out_shape=(
            jax.ShapeDtypeStruct((n, h2), jnp.float32),
            jax.ShapeDtypeStruct((n, h2), jnp.float32),
        ),
    )(dinv_col, s1, xw, b1_row, w2)


def _final_kernel(n, dinv_ref, s2_ref, hw2_ref, b2_ref, o_ref):
    dinv = dinv_ref[...]
    s2 = s2_ref[0, :n] + s2_ref[1, :n]
    o_ref[...] = dinv * s2 + dinv * dinv * hw2_ref[...] + b2_ref[...]


def _tc_final(dinv_col, s2, hw2, b2_row):
    return pl.pallas_call(
        functools.partial(_final_kernel, hw2.shape[0]),
        out_shape=jax.ShapeDtypeStruct(hw2.shape, jnp.float32),
    )(dinv_col, s2, hw2, b2_row)


@jax.jit
def kernel(x, edge_index, edge_weight, W1, b1, W2, b2):
    n, _ = x.shape
    e = edge_weight.shape[0]

    n_chunks = -(-e // (NW * CHUNK))
    e_pad = NW * CHUNK * n_chunks
    pad = e_pad - e
    n_pad = -(-n // (NS * SUB)) * (NS * SUB)

    fill = jnp.arange(pad, dtype=jnp.int32) % n
    src_p = jnp.concatenate([edge_index[0], fill]).reshape(e_pad // SUB, SUB)
    dst_p = jnp.concatenate([edge_index[1], fill]).reshape(e_pad // SUB, SUB)
    ew_p = jnp.concatenate(
        [edge_weight, jnp.zeros((pad,), jnp.float32)])

    xw = _tc_matmul(x, W1)
    degp = _sc_deg(dst_p, ew_p, n_pad, n_chunks)

    y1, dinv_col = _tc_scale(degp, xw)
    s1 = _sc_agg(y1, src_p, dst_p, ew_p, W1.shape[1], n_pad, n_chunks)
    hw2, y2 = _tc_mid(dinv_col, s1, xw, b1.reshape(1, -1), W2)
    s2 = _sc_agg(y2, src_p, dst_p, ew_p, W2.shape[1], n_pad, n_chunks)
    return _tc_final(dinv_col, s2, hw2, b2.reshape(1, -1))


# speedup vs baseline: 50.5669x; 1.1098x over previous
import functools

import jax
import jax.numpy as jnp
from jax import lax
from jax.experimental import pallas as pl
from jax.experimental.pallas import tpu as pltpu
from jax.experimental.pallas import tpu_sc as plsc

NC = 2
NS = 16
NW = NC * NS
LANES = 16
SUB = 128
CHUNK = 1024


def _mesh():
    return plsc.VectorSubcoreMesh(core_axis_name="c", subcore_axis_name="s")


def _sc_deg(dst_flat, ew_flat, n_pad, n_chunks):
    slice_n = n_pad // NS

    @functools.partial(
        pl.kernel,
        out_type=jax.ShapeDtypeStruct((NC, n_pad), jnp.float32),
        mesh=_mesh(),
        scratch_types=[
            pltpu.VMEM((CHUNK // SUB, SUB), jnp.int32),
            pltpu.VMEM((CHUNK,), jnp.float32),
            pltpu.VMEM((slice_n,), jnp.float32),
            pltpu.VMEM_SHARED((n_pad,), jnp.float32),
            pltpu.SemaphoreType.DMA,
        ],
    )
    def k(dst_hbm, ew_hbm, out_hbm, dstv, eww, zbuf, acc, sem):
        c = lax.axis_index("c")
        s = lax.axis_index("s")
        wid = c * NS + s

        @pl.loop(0, slice_n // LANES)
        def _(i):
            zbuf[pl.ds(i * LANES, LANES)] = jnp.zeros((LANES,), jnp.float32)

        pltpu.sync_copy(zbuf, acc.at[pl.ds(s * slice_n, slice_n)])
        plsc.subcore_barrier()

        @pl.loop(0, n_chunks)
        def _(ch):
            base = wid * (n_chunks * CHUNK) + ch * CHUNK
            for j in range(CHUNK // SUB):
                pltpu.async_copy(dst_hbm.at[pl.ds(base + j * SUB, SUB)],
                                 dstv.at[j], sem)
            pltpu.async_copy(ew_hbm.at[pl.ds(base, CHUNK)], eww, sem)
            for j in range(CHUNK // SUB):
                pltpu.make_async_copy(dst_hbm.at[pl.ds(base + j * SUB, SUB)],
                                      dstv.at[j], sem).wait()
            pltpu.make_async_copy(ew_hbm.at[pl.ds(base, CHUNK)], eww,
                                  sem).wait()
            cps = [
                pltpu.async_copy(
                    eww.at[pl.ds(j * SUB, SUB)], acc.at[dstv.at[j]], sem,
                    add=True)
                for j in range(CHUNK // SUB)
            ]
            for cp in cps:
                cp.wait()

        plsc.subcore_barrier()
        pltpu.sync_copy(acc.at[pl.ds(s * slice_n, slice_n)],
                        out_hbm.at[c].at[pl.ds(s * slice_n, slice_n)])

    return k(dst_flat, ew_flat)


def _sc_agg(y, src_flat, dst_flat, ew_flat, h, n_pad, n_chunks):
    slice_n = n_pad // NS
    nsub = CHUNK // SUB
    assert n_chunks % 2 == 0

    @functools.partial(
        pl.kernel,
        out_type=jax.ShapeDtypeStruct((NC, n_pad, h), jnp.float32),
        mesh=_mesh(),
        scratch_types=[
            pltpu.VMEM((2, nsub, SUB), jnp.int32),
            pltpu.VMEM((2, nsub, SUB), jnp.int32),
            pltpu.VMEM((2, CHUNK), jnp.float32),
            pltpu.VMEM((2, CHUNK, h), jnp.float32),
            pltpu.VMEM_SHARED((n_pad, h), jnp.float32),
            pltpu.SemaphoreType.DMA,
            pltpu.SemaphoreType.DMA,
            pltpu.SemaphoreType.DMA,
            pltpu.SemaphoreType.DMA,
        ],
        compiler_params=pltpu.CompilerParams(use_tc_tiling_on_sc=False),
    )
    def k(y_hbm, src_hbm, dst_hbm, ew_hbm, out_hbm,
          srcv, dstv, eww, rows, acc, gsem0, gsem1, ssem0, ssem1):
        c = lax.axis_index("c")
        s = lax.axis_index("s")
        wid = c * NS + s
        gsem = (gsem0, gsem1)
        ssem = (ssem0, ssem1)

        @pl.loop(0, slice_n)
        def _(i):
            for kk in range(h // LANES):
                rows[0, i, pl.ds(kk * LANES, LANES)] = jnp.zeros(
                    (LANES,), jnp.float32)

        pltpu.sync_copy(rows.at[0, pl.ds(0, slice_n)],
                        acc.at[pl.ds(s * slice_n, slice_n)])
        plsc.subcore_barrier()

        def load_idx(ch, b):
            base = wid * (n_chunks * CHUNK) + ch * CHUNK
            for j in range(nsub):
                pltpu.async_copy(src_hbm.at[pl.ds(base + j * SUB, SUB)],
                                 srcv.at[b].at[j], gsem[b])
                pltpu.async_copy(dst_hbm.at[pl.ds(base + j * SUB, SUB)],
                                 dstv.at[b].at[j], gsem[b])
            pltpu.async_copy(ew_hbm.at[pl.ds(base, CHUNK)], eww.at[b],
                             gsem[b])
            for j in range(nsub):
                pltpu.make_async_copy(src_hbm.at[pl.ds(base + j * SUB, SUB)],
                                      srcv.at[b].at[j], gsem[b]).wait()
                pltpu.make_async_copy(dst_hbm.at[pl.ds(base + j * SUB, SUB)],
                                      dstv.at[b].at[j], gsem[b]).wait()
            pltpu.make_async_copy(ew_hbm.at[pl.ds(base, CHUNK)], eww.at[b],
                                  gsem[b]).wait()

        def fire_gather(b):
            for j in range(nsub):
                pltpu.async_copy(y_hbm.at[srcv.at[b].at[j]],
                                 rows.at[b].at[pl.ds(j * SUB, SUB)], gsem[b])

        def wait_gather(b):
            for j in range(nsub):
                pltpu.make_async_copy(
                    y_hbm.at[srcv.at[b].at[j]],
                    rows.at[b].at[pl.ds(j * SUB, SUB)], gsem[b]).wait()

        def fire_scatter(b):
            for j in range(nsub):
                pltpu.async_copy(rows.at[b].at[pl.ds(j * SUB, SUB)],
                                 acc.at[dstv.at[b].at[j]], ssem[b], add=True)

        def wait_scatter(b):
            for j in range(nsub):
                pltpu.make_async_copy(
                    rows.at[b].at[pl.ds(j * SUB, SUB)],
                    acc.at[dstv.at[b].at[j]], ssem[b]).wait()

        def compute(b):
            @pl.loop(0, CHUNK // LANES)
            def _(g):
                wreg = eww.at[b][pl.ds(g * LANES, LANES)]
                for j in range(LANES):
                    e = g * LANES + j
                    wj = wreg[j]
                    for kk in range(h // LANES):
                        sl = pl.ds(kk * LANES, LANES)
                        rows[b, e, sl] = rows[b, e, sl] * wj

        load_idx(0, 0)
        fire_gather(0)

        @pl.loop(0, n_chunks // 2)
        def _(i):
            c0 = 2 * i

            @pl.when(i > 0)
            def _():
                wait_scatter(1)
            load_idx(c0 + 1, 1)
            fire_gather(1)

            wait_gather(0)
            compute(0)
            fire_scatter(0)

            wait_scatter(0)

            @pl.when(c0 + 2 < n_chunks)
            def _():
                load_idx(c0 + 2, 0)
                fire_gather(0)

            wait_gather(1)
            compute(1)
            fire_scatter(1)

        wait_scatter(1)
        plsc.subcore_barrier()
        pltpu.sync_copy(acc.at[pl.ds(s * slice_n, slice_n)],
                        out_hbm.at[c].at[pl.ds(s * slice_n, slice_n)])

    return k(y, src_flat, dst_flat, ew_flat)


def _mm_kernel(x_ref, w_ref, o_ref):
    o_ref[...] = jnp.dot(x_ref[...], w_ref[...])


def _tc_matmul(x, w):
    return pl.pallas_call(
        _mm_kernel,
        out_shape=jax.ShapeDtypeStruct((x.shape[0], w.shape[1]), jnp.float32),
    )(x, w)


def _scale_kernel(n, degp_ref, xw_ref, y_ref, dinv_ref):
    deg = degp_ref[0] + degp_ref[1] + 1.0
    dinv = jnp.where(deg > 0, lax.rsqrt(deg), 0.0)
    dinv_col = dinv.reshape(deg.shape[0], 1)[:n]
    y_ref[...] = xw_ref[...] * dinv_col
    dinv_ref[...] = dinv_col


def _tc_scale(degp, xw):
    n = xw.shape[0]
    return pl.pallas_call(
        functools.partial(_scale_kernel, n),
        out_shape=(
            jax.ShapeDtypeStruct(xw.shape, jnp.float32),
            jax.ShapeDtypeStruct((n, 1), jnp.float32),
        ),
    )(degp, xw)


def _mid_kernel(n, dinv_ref, s1_ref, xw_ref, b1_ref, w2_ref,
                hw2_ref, y2_ref):
    dinv = dinv_ref[...]
    s1 = s1_ref[0, :n] + s1_ref[1, :n]
    h = jnp.maximum(dinv * s1 + dinv * dinv * xw_ref[...] + b1_ref[...], 0.0)
    hw2 = jnp.dot(h, w2_ref[...])
    hw2_ref[...] = hw2
    y2_ref[...] = hw2 * dinv


def _tc_mid(dinv_col, s1, xw, b1_row, w2):
    n = xw.shape[0]
    h2 = w2.shape[1]
    return pl.pallas_call(
        functools.partial(_mid_kernel, n),
        out_shape=(
            jax.ShapeDtypeStruct((n, h2), jnp.float32),
            jax.ShapeDtypeStruct((n, h2), jnp.float32),
        ),
    )(dinv_col, s1, xw, b1_row, w2)


def _final_kernel(n, dinv_ref, s2_ref, hw2_ref, b2_ref, o_ref):
    dinv = dinv_ref[...]
    s2 = s2_ref[0, :n] + s2_ref[1, :n]
    o_ref[...] = dinv * s2 + dinv * dinv * hw2_ref[...] + b2_ref[...]


def _tc_final(dinv_col, s2, hw2, b2_row):
    return pl.pallas_call(
        functools.partial(_final_kernel, hw2.shape[0]),
        out_shape=jax.ShapeDtypeStruct(hw2.shape, jnp.float32),
    )(dinv_col, s2, hw2, b2_row)


@jax.jit
def kernel(x, edge_index, edge_weight, W1, b1, W2, b2):
    n, _ = x.shape
    e = edge_weight.shape[0]

    n_chunks = -(-e // (NW * CHUNK))
    e_pad = NW * CHUNK * n_chunks
    pad = e_pad - e
    n_pad = -(-n // (NS * SUB)) * (NS * SUB)

    fill = jnp.arange(pad, dtype=jnp.int32) % n
    src_p = jnp.concatenate([edge_index[0], fill])
    dst_p = jnp.concatenate([edge_index[1], fill])
    ew_p = jnp.concatenate(
        [edge_weight, jnp.zeros((pad,), jnp.float32)])

    xw = _tc_matmul(x, W1)
    degp = _sc_deg(dst_p, ew_p, n_pad, n_chunks)

    y1, dinv_col = _tc_scale(degp, xw)
    s1 = _sc_agg(y1, src_p, dst_p, ew_p, W1.shape[1], n_pad, n_chunks)
    hw2, y2 = _tc_mid(dinv_col, s1, xw, b1.reshape(1, -1), W2)
    s2 = _sc_agg(y2, src_p, dst_p, ew_p, W2.shape[1], n_pad, n_chunks)
    return _tc_final(dinv_col, s2, hw2, b2.reshape(1, -1))


# speedup vs baseline: 51.0414x; 1.0094x over previous
import functools

import jax
import jax.numpy as jnp
from jax import lax
from jax.experimental import pallas as pl
from jax.experimental.pallas import tpu as pltpu
from jax.experimental.pallas import tpu_sc as plsc

NC = 2
NS = 16
NW = NC * NS
LANES = 16
SUB = 128
CHUNK = 1024


def _mesh():
    return plsc.VectorSubcoreMesh(core_axis_name="c", subcore_axis_name="s")


def _sc_deg(dst_flat, ew_flat, n_pad, n_chunks):
    slice_n = n_pad // NS

    @functools.partial(
        pl.kernel,
        out_type=jax.ShapeDtypeStruct((NC, n_pad), jnp.float32),
        mesh=_mesh(),
        scratch_types=[
            pltpu.VMEM((CHUNK // SUB, SUB), jnp.int32),
            pltpu.VMEM((CHUNK,), jnp.float32),
            pltpu.VMEM((slice_n,), jnp.float32),
            pltpu.VMEM_SHARED((n_pad,), jnp.float32),
            pltpu.SemaphoreType.DMA,
        ],
    )
    def k(dst_hbm, ew_hbm, out_hbm, dstv, eww, zbuf, acc, sem):
        c = lax.axis_index("c")
        s = lax.axis_index("s")
        wid = c * NS + s

        @pl.loop(0, slice_n // LANES)
        def _(i):
            zbuf[pl.ds(i * LANES, LANES)] = jnp.zeros((LANES,), jnp.float32)

        pltpu.sync_copy(zbuf, acc.at[pl.ds(s * slice_n, slice_n)])
        plsc.subcore_barrier()

        @pl.loop(0, n_chunks)
        def _(ch):
            base = wid * (n_chunks * CHUNK) + ch * CHUNK
            for j in range(CHUNK // SUB):
                pltpu.async_copy(dst_hbm.at[pl.ds(base + j * SUB, SUB)],
                                 dstv.at[j], sem)
            pltpu.async_copy(ew_hbm.at[pl.ds(base, CHUNK)], eww, sem)
            for j in range(CHUNK // SUB):
                pltpu.make_async_copy(dst_hbm.at[pl.ds(base + j * SUB, SUB)],
                                      dstv.at[j], sem).wait()
            pltpu.make_async_copy(ew_hbm.at[pl.ds(base, CHUNK)], eww,
                                  sem).wait()
            cps = [
                pltpu.async_copy(
                    eww.at[pl.ds(j * SUB, SUB)], acc.at[dstv.at[j]], sem,
                    add=True)
                for j in range(CHUNK // SUB)
            ]
            for cp in cps:
                cp.wait()

        plsc.subcore_barrier()
        pltpu.sync_copy(acc.at[pl.ds(s * slice_n, slice_n)],
                        out_hbm.at[c].at[pl.ds(s * slice_n, slice_n)])

    return k(dst_flat, ew_flat)


def _sc_agg(y, src_flat, dst_flat, ew_flat, h, n_pad, n_chunks):
    slice_n = n_pad // NS
    nsub = CHUNK // SUB
    assert n_chunks % 2 == 0

    @functools.partial(
        pl.kernel,
        out_type=jax.ShapeDtypeStruct((NC, n_pad, h), jnp.float32),
        mesh=_mesh(),
        scratch_types=[
            pltpu.VMEM((2, nsub, SUB), jnp.int32),
            pltpu.VMEM((2, nsub, SUB), jnp.int32),
            pltpu.VMEM((2, CHUNK), jnp.float32),
            pltpu.VMEM((2, CHUNK, h), jnp.float32),
            pltpu.VMEM_SHARED((n_pad, h), jnp.float32),
            pltpu.SemaphoreType.DMA,
            pltpu.SemaphoreType.DMA,
            pltpu.SemaphoreType.DMA,
            pltpu.SemaphoreType.DMA,
        ],
        compiler_params=pltpu.CompilerParams(use_tc_tiling_on_sc=False),
    )
    def k(y_hbm, src_hbm, dst_hbm, ew_hbm, out_hbm,
          srcv, dstv, eww, rows, acc, gsem0, gsem1, ssem0, ssem1):
        c = lax.axis_index("c")
        s = lax.axis_index("s")
        wid = c * NS + s
        gsem = (gsem0, gsem1)
        ssem = (ssem0, ssem1)

        @pl.loop(0, slice_n)
        def _(i):
            for kk in range(h // LANES):
                rows[0, i, pl.ds(kk * LANES, LANES)] = jnp.zeros(
                    (LANES,), jnp.float32)

        pltpu.sync_copy(rows.at[0, pl.ds(0, slice_n)],
                        acc.at[pl.ds(s * slice_n, slice_n)])
        plsc.subcore_barrier()

        def load_idx(ch, b):
            base = wid * (n_chunks * CHUNK) + ch * CHUNK
            for j in range(nsub):
                pltpu.async_copy(src_hbm.at[pl.ds(base + j * SUB, SUB)],
                                 srcv.at[b].at[j], gsem[b])
                pltpu.async_copy(dst_hbm.at[pl.ds(base + j * SUB, SUB)],
                                 dstv.at[b].at[j], gsem[b])
            pltpu.async_copy(ew_hbm.at[pl.ds(base, CHUNK)], eww.at[b],
                             gsem[b])
            for j in range(nsub):
                pltpu.make_async_copy(src_hbm.at[pl.ds(base + j * SUB, SUB)],
                                      srcv.at[b].at[j], gsem[b]).wait()
                pltpu.make_async_copy(dst_hbm.at[pl.ds(base + j * SUB, SUB)],
                                      dstv.at[b].at[j], gsem[b]).wait()
            pltpu.make_async_copy(ew_hbm.at[pl.ds(base, CHUNK)], eww.at[b],
                                  gsem[b]).wait()

        def fire_gather(b):
            for j in range(nsub):
                pltpu.async_copy(y_hbm.at[srcv.at[b].at[j]],
                                 rows.at[b].at[pl.ds(j * SUB, SUB)], gsem[b])

        def wait_gather(b):
            for j in range(nsub):
                pltpu.make_async_copy(
                    y_hbm.at[srcv.at[b].at[j]],
                    rows.at[b].at[pl.ds(j * SUB, SUB)], gsem[b]).wait()

        def fire_scatter(b):
            for j in range(nsub):
                pltpu.async_copy(rows.at[b].at[pl.ds(j * SUB, SUB)],
                                 acc.at[dstv.at[b].at[j]], ssem[b], add=True)

        def wait_scatter(b):
            for j in range(nsub):
                pltpu.make_async_copy(
                    rows.at[b].at[pl.ds(j * SUB, SUB)],
                    acc.at[dstv.at[b].at[j]], ssem[b]).wait()

        def compute(b):
            @pl.loop(0, CHUNK // LANES)
            def _(g):
                wreg = eww.at[b][pl.ds(g * LANES, LANES)]
                for j in range(LANES):
                    e = g * LANES + j
                    wj = wreg[j]
                    for kk in range(h // LANES):
                        sl = pl.ds(kk * LANES, LANES)
                        rows[b, e, sl] = rows[b, e, sl] * wj

        load_idx(0, 0)
        fire_gather(0)
        load_idx(1, 1)
        fire_gather(1)

        @pl.loop(0, n_chunks // 2)
        def _(i):
            c0 = 2 * i

            wait_gather(0)
            compute(0)
            fire_scatter(0)

            wait_gather(1)
            compute(1)
            fire_scatter(1)

            wait_scatter(0)

            @pl.when(c0 + 2 < n_chunks)
            def _():
                load_idx(c0 + 2, 0)
                fire_gather(0)

            wait_scatter(1)

            @pl.when(c0 + 3 < n_chunks)
            def _():
                load_idx(c0 + 3, 1)
                fire_gather(1)
        plsc.subcore_barrier()
        pltpu.sync_copy(acc.at[pl.ds(s * slice_n, slice_n)],
                        out_hbm.at[c].at[pl.ds(s * slice_n, slice_n)])

    return k(y, src_flat, dst_flat, ew_flat)


def _mm_kernel(x_ref, w_ref, o_ref):
    o_ref[...] = jnp.dot(x_ref[...], w_ref[...])


def _tc_matmul(x, w):
    return pl.pallas_call(
        _mm_kernel,
        out_shape=jax.ShapeDtypeStruct((x.shape[0], w.shape[1]), jnp.float32),
    )(x, w)


def _scale_kernel(n, degp_ref, xw_ref, y_ref, dinv_ref):
    deg = degp_ref[0] + degp_ref[1] + 1.0
    dinv = jnp.where(deg > 0, lax.rsqrt(deg), 0.0)
    dinv_col = dinv.reshape(deg.shape[0], 1)[:n]
    y_ref[...] = xw_ref[...] * dinv_col
    dinv_ref[...] = dinv_col


def _tc_scale(degp, xw):
    n = xw.shape[0]
    return pl.pallas_call(
        functools.partial(_scale_kernel, n),
        out_shape=(
            jax.ShapeDtypeStruct(xw.shape, jnp.float32),
            jax.ShapeDtypeStruct((n, 1), jnp.float32),
        ),
    )(degp, xw)


def _mid_kernel(n, dinv_ref, s1_ref, xw_ref, b1_ref, w2_ref,
                hw2_ref, y2_ref):
    dinv = dinv_ref[...]
    s1 = s1_ref[0, :n] + s1_ref[1, :n]
    h = jnp.maximum(dinv * s1 + dinv * dinv * xw_ref[...] + b1_ref[...], 0.0)
    hw2 = jnp.dot(h, w2_ref[...])
    hw2_ref[...] = hw2
    y2_ref[...] = hw2 * dinv


def _tc_mid(dinv_col, s1, xw, b1_row, w2):
    n = xw.shape[0]
    h2 = w2.shape[1]
    return pl.pallas_call(
        functools.partial(_mid_kernel, n),
        out_shape=(
            jax.ShapeDtypeStruct((n, h2), jnp.float32),
            jax.ShapeDtypeStruct((n, h2), jnp.float32),
        ),
    )(dinv_col, s1, xw, b1_row, w2)


def _final_kernel(n, dinv_ref, s2_ref, hw2_ref, b2_ref, o_ref):
    dinv = dinv_ref[...]
    s2 = s2_ref[0, :n] + s2_ref[1, :n]
    o_ref[...] = dinv * s2 + dinv * dinv * hw2_ref[...] + b2_ref[...]


def _tc_final(dinv_col, s2, hw2, b2_row):
    return pl.pallas_call(
        functools.partial(_final_kernel, hw2.shape[0]),
        out_shape=jax.ShapeDtypeStruct(hw2.shape, jnp.float32),
    )(dinv_col, s2, hw2, b2_row)


@jax.jit
def kernel(x, edge_index, edge_weight, W1, b1, W2, b2):
    n, _ = x.shape
    e = edge_weight.shape[0]

    n_chunks = -(-e // (NW * CHUNK))
    e_pad = NW * CHUNK * n_chunks
    pad = e_pad - e
    n_pad = -(-n // (NS * SUB)) * (NS * SUB)

    fill = jnp.arange(pad, dtype=jnp.int32) % n
    src_p = jnp.concatenate([edge_index[0], fill])
    dst_p = jnp.concatenate([edge_index[1], fill])
    ew_p = jnp.concatenate(
        [edge_weight, jnp.zeros((pad,), jnp.float32)])

    xw = _tc_matmul(x, W1)
    degp = _sc_deg(dst_p, ew_p, n_pad, n_chunks)

    y1, dinv_col = _tc_scale(degp, xw)
    s1 = _sc_agg(y1, src_p, dst_p, ew_p, W1.shape[1], n_pad, n_chunks)
    hw2, y2 = _tc_mid(dinv_col, s1, xw, b1.reshape(1, -1), W2)
    s2 = _sc_agg(y2, src_p, dst_p, ew_p, W2.shape[1], n_pad, n_chunks)
    return _tc_final(dinv_col, s2, hw2, b2.reshape(1, -1))


# speedup vs baseline: 52.1595x; 1.0219x over previous
import functools

import jax
import jax.numpy as jnp
from jax import lax
from jax.experimental import pallas as pl
from jax.experimental.pallas import tpu as pltpu
from jax.experimental.pallas import tpu_sc as plsc

NC = 2
NS = 16
NW = NC * NS
LANES = 16
SUB = 128
CHUNK = 1024


def _mesh():
    return plsc.VectorSubcoreMesh(core_axis_name="c", subcore_axis_name="s")


def _sc_deg(dst_flat, ew_flat, n_pad, n_chunks):
    slice_n = n_pad // NS

    nsub = CHUNK // SUB
    assert n_chunks % 2 == 0

    @functools.partial(
        pl.kernel,
        out_type=jax.ShapeDtypeStruct((NC, n_pad), jnp.float32),
        mesh=_mesh(),
        scratch_types=[
            pltpu.VMEM((2, nsub, SUB), jnp.int32),
            pltpu.VMEM((2, CHUNK), jnp.float32),
            pltpu.VMEM((slice_n,), jnp.float32),
            pltpu.VMEM_SHARED((n_pad,), jnp.float32),
            pltpu.SemaphoreType.DMA,
            pltpu.SemaphoreType.DMA,
            pltpu.SemaphoreType.DMA,
            pltpu.SemaphoreType.DMA,
        ],
    )
    def k(dst_hbm, ew_hbm, out_hbm, dstv, eww, zbuf, acc,
          lsem0, lsem1, ssem0, ssem1):
        c = lax.axis_index("c")
        s = lax.axis_index("s")
        wid = c * NS + s
        lsem = (lsem0, lsem1)
        ssem = (ssem0, ssem1)

        @pl.loop(0, slice_n // LANES)
        def _(i):
            zbuf[pl.ds(i * LANES, LANES)] = jnp.zeros((LANES,), jnp.float32)

        pltpu.sync_copy(zbuf, acc.at[pl.ds(s * slice_n, slice_n)])
        plsc.subcore_barrier()

        def load(ch, b):
            base = wid * (n_chunks * CHUNK) + ch * CHUNK
            for j in range(nsub):
                pltpu.async_copy(dst_hbm.at[pl.ds(base + j * SUB, SUB)],
                                 dstv.at[b].at[j], lsem[b])
            pltpu.async_copy(ew_hbm.at[pl.ds(base, CHUNK)], eww.at[b],
                             lsem[b])

        def wait_load(ch, b):
            base = wid * (n_chunks * CHUNK) + ch * CHUNK
            for j in range(nsub):
                pltpu.make_async_copy(dst_hbm.at[pl.ds(base + j * SUB, SUB)],
                                      dstv.at[b].at[j], lsem[b]).wait()
            pltpu.make_async_copy(ew_hbm.at[pl.ds(base, CHUNK)], eww.at[b],
                                  lsem[b]).wait()

        def fire_scat(b):
            for j in range(nsub):
                pltpu.async_copy(eww.at[b].at[pl.ds(j * SUB, SUB)],
                                 acc.at[dstv.at[b].at[j]], ssem[b], add=True)

        def wait_scat(b):
            for j in range(nsub):
                pltpu.make_async_copy(
                    eww.at[b].at[pl.ds(j * SUB, SUB)],
                    acc.at[dstv.at[b].at[j]], ssem[b]).wait()

        load(0, 0)
        load(1, 1)

        @pl.loop(0, n_chunks // 2)
        def _(i):
            c0 = 2 * i

            wait_load(c0, 0)
            fire_scat(0)
            wait_load(c0 + 1, 1)
            fire_scat(1)

            wait_scat(0)

            @pl.when(c0 + 2 < n_chunks)
            def _():
                load(c0 + 2, 0)

            wait_scat(1)

            @pl.when(c0 + 3 < n_chunks)
            def _():
                load(c0 + 3, 1)

        plsc.subcore_barrier()
        pltpu.sync_copy(acc.at[pl.ds(s * slice_n, slice_n)],
                        out_hbm.at[c].at[pl.ds(s * slice_n, slice_n)])

    return k(dst_flat, ew_flat)


def _sc_agg(y, src_flat, dst_flat, ew_flat, h, n_pad, n_chunks):
    slice_n = n_pad // NS
    nsub = CHUNK // SUB
    assert n_chunks % 2 == 0

    @functools.partial(
        pl.kernel,
        out_type=jax.ShapeDtypeStruct((NC, n_pad, h), jnp.float32),
        mesh=_mesh(),
        scratch_types=[
            pltpu.VMEM((2, nsub, SUB), jnp.int32),
            pltpu.VMEM((2, nsub, SUB), jnp.int32),
            pltpu.VMEM((2, CHUNK), jnp.float32),
            pltpu.VMEM((2, CHUNK, h), jnp.float32),
            pltpu.VMEM_SHARED((n_pad, h), jnp.float32),
            pltpu.SemaphoreType.DMA,
            pltpu.SemaphoreType.DMA,
            pltpu.SemaphoreType.DMA,
            pltpu.SemaphoreType.DMA,
        ],
        compiler_params=pltpu.CompilerParams(use_tc_tiling_on_sc=False),
    )
    def k(y_hbm, src_hbm, dst_hbm, ew_hbm, out_hbm,
          srcv, dstv, eww, rows, acc, gsem0, gsem1, ssem0, ssem1):
        c = lax.axis_index("c")
        s = lax.axis_index("s")
        wid = c * NS + s
        gsem = (gsem0, gsem1)
        ssem = (ssem0, ssem1)

        @pl.loop(0, slice_n)
        def _(i):
            for kk in range(h // LANES):
                rows[0, i, pl.ds(kk * LANES, LANES)] = jnp.zeros(
                    (LANES,), jnp.float32)

        pltpu.sync_copy(rows.at[0, pl.ds(0, slice_n)],
                        acc.at[pl.ds(s * slice_n, slice_n)])
        plsc.subcore_barrier()

        def load_idx(ch, b):
            base = wid * (n_chunks * CHUNK) + ch * CHUNK
            for j in range(nsub):
                pltpu.async_copy(src_hbm.at[pl.ds(base + j * SUB, SUB)],
                                 srcv.at[b].at[j], gsem[b])
                pltpu.async_copy(dst_hbm.at[pl.ds(base + j * SUB, SUB)],
                                 dstv.at[b].at[j], gsem[b])
            pltpu.async_copy(ew_hbm.at[pl.ds(base, CHUNK)], eww.at[b],
                             gsem[b])
            for j in range(nsub):
                pltpu.make_async_copy(src_hbm.at[pl.ds(base + j * SUB, SUB)],
                                      srcv.at[b].at[j], gsem[b]).wait()
                pltpu.make_async_copy(dst_hbm.at[pl.ds(base + j * SUB, SUB)],
                                      dstv.at[b].at[j], gsem[b]).wait()
            pltpu.make_async_copy(ew_hbm.at[pl.ds(base, CHUNK)], eww.at[b],
                                  gsem[b]).wait()

        def fire_gather(b):
            for j in range(nsub):
                pltpu.async_copy(y_hbm.at[srcv.at[b].at[j]],
                                 rows.at[b].at[pl.ds(j * SUB, SUB)], gsem[b])

        def wait_gather(b):
            for j in range(nsub):
                pltpu.make_async_copy(
                    y_hbm.at[srcv.at[b].at[j]],
                    rows.at[b].at[pl.ds(j * SUB, SUB)], gsem[b]).wait()

        def fire_scatter(b):
            for j in range(nsub):
                pltpu.async_copy(rows.at[b].at[pl.ds(j * SUB, SUB)],
                                 acc.at[dstv.at[b].at[j]], ssem[b], add=True)

        def wait_scatter(b):
            for j in range(nsub):
                pltpu.make_async_copy(
                    rows.at[b].at[pl.ds(j * SUB, SUB)],
                    acc.at[dstv.at[b].at[j]], ssem[b]).wait()

        def compute(b):
            @pl.loop(0, CHUNK // LANES)
            def _(g):
                wreg = eww.at[b][pl.ds(g * LANES, LANES)]
                for j in range(LANES):
                    e = g * LANES + j
                    wj = wreg[j]
                    for kk in range(h // LANES):
                        sl = pl.ds(kk * LANES, LANES)
                        rows[b, e, sl] = rows[b, e, sl] * wj

        load_idx(0, 0)
        fire_gather(0)
        load_idx(1, 1)
        fire_gather(1)

        @pl.loop(0, n_chunks // 2)
        def _(i):
            c0 = 2 * i

            wait_gather(0)
            compute(0)
            fire_scatter(0)

            wait_gather(1)
            compute(1)
            fire_scatter(1)

            wait_scatter(0)

            @pl.when(c0 + 2 < n_chunks)
            def _():
                load_idx(c0 + 2, 0)
                fire_gather(0)

            wait_scatter(1)

            @pl.when(c0 + 3 < n_chunks)
            def _():
                load_idx(c0 + 3, 1)
                fire_gather(1)
        plsc.subcore_barrier()
        pltpu.sync_copy(acc.at[pl.ds(s * slice_n, slice_n)],
                        out_hbm.at[c].at[pl.ds(s * slice_n, slice_n)])

    return k(y, src_flat, dst_flat, ew_flat)


def _mm_kernel(x_ref, w_ref, o_ref):
    o_ref[...] = jnp.dot(x_ref[...], w_ref[...])


def _tc_matmul(x, w):
    return pl.pallas_call(
        _mm_kernel,
        out_shape=jax.ShapeDtypeStruct((x.shape[0], w.shape[1]), jnp.float32),
    )(x, w)


def _scale_kernel(n, degp_ref, xw_ref, y_ref, dinv_ref):
    deg = degp_ref[0] + degp_ref[1] + 1.0
    dinv = jnp.where(deg > 0, lax.rsqrt(deg), 0.0)
    dinv_col = dinv.reshape(deg.shape[0], 1)[:n]
    y_ref[...] = xw_ref[...] * dinv_col
    dinv_ref[...] = dinv_col


def _tc_scale(degp, xw):
    n = xw.shape[0]
    return pl.pallas_call(
        functools.partial(_scale_kernel, n),
        out_shape=(
            jax.ShapeDtypeStruct(xw.shape, jnp.float32),
            jax.ShapeDtypeStruct((n, 1), jnp.float32),
        ),
    )(degp, xw)


def _mid_kernel(n, dinv_ref, s1_ref, xw_ref, b1_ref, w2_ref,
                hw2_ref, y2_ref):
    dinv = dinv_ref[...]
    s1 = s1_ref[0, :n] + s1_ref[1, :n]
    h = jnp.maximum(dinv * s1 + dinv * dinv * xw_ref[...] + b1_ref[...], 0.0)
    hw2 = jnp.dot(h, w2_ref[...])
    hw2_ref[...] = hw2
    y2_ref[...] = hw2 * dinv


def _tc_mid(dinv_col, s1, xw, b1_row, w2):
    n = xw.shape[0]
    h2 = w2.shape[1]
    return pl.pallas_call(
        functools.partial(_mid_kernel, n),
        out_shape=(
            jax.ShapeDtypeStruct((n, h2), jnp.float32),
            jax.ShapeDtypeStruct((n, h2), jnp.float32),
        ),
    )(dinv_col, s1, xw, b1_row, w2)


def _final_kernel(n, dinv_ref, s2_ref, hw2_ref, b2_ref, o_ref):
    dinv = dinv_ref[...]
    s2 = s2_ref[0, :n] + s2_ref[1, :n]
    o_ref[...] = dinv * s2 + dinv * dinv * hw2_ref[...] + b2_ref[...]


def _tc_final(dinv_col, s2, hw2, b2_row):
    return pl.pallas_call(
        functools.partial(_final_kernel, hw2.shape[0]),
        out_shape=jax.ShapeDtypeStruct(hw2.shape, jnp.float32),
    )(dinv_col, s2, hw2, b2_row)


@jax.jit
def kernel(x, edge_index, edge_weight, W1, b1, W2, b2):
    n, _ = x.shape
    e = edge_weight.shape[0]

    n_chunks = -(-e // (NW * CHUNK))
    e_pad = NW * CHUNK * n_chunks
    pad = e_pad - e
    n_pad = -(-n // (NS * SUB)) * (NS * SUB)

    fill = jnp.arange(pad, dtype=jnp.int32) % n
    src_p = jnp.concatenate([edge_index[0], fill])
    dst_p = jnp.concatenate([edge_index[1], fill])
    ew_p = jnp.concatenate(
        [edge_weight, jnp.zeros((pad,), jnp.float32)])

    xw = _tc_matmul(x, W1)
    degp = _sc_deg(dst_p, ew_p, n_pad, n_chunks)

    y1, dinv_col = _tc_scale(degp, xw)
    s1 = _sc_agg(y1, src_p, dst_p, ew_p, W1.shape[1], n_pad, n_chunks)
    hw2, y2 = _tc_mid(dinv_col, s1, xw, b1.reshape(1, -1), W2)
    s2 = _sc_agg(y2, src_p, dst_p, ew_p, W2.shape[1], n_pad, n_chunks)
    return _tc_final(dinv_col, s2, hw2, b2.reshape(1, -1))


# speedup vs baseline: 52.3390x; 1.0034x over previous
import functools

import jax
import jax.numpy as jnp
from jax import lax
from jax.experimental import pallas as pl
from jax.experimental.pallas import tpu as pltpu
from jax.experimental.pallas import tpu_sc as plsc

NC = 2
NS = 16
NW = NC * NS
LANES = 16
SUB = 128
CHUNK = 1280


def _mesh():
    return plsc.VectorSubcoreMesh(core_axis_name="c", subcore_axis_name="s")


def _sc_deg(dst_flat, ew_flat, n_pad, n_chunks):
    slice_n = n_pad // NS

    nsub = CHUNK // SUB
    assert n_chunks % 2 == 0

    @functools.partial(
        pl.kernel,
        out_type=jax.ShapeDtypeStruct((NC, n_pad), jnp.float32),
        mesh=_mesh(),
        scratch_types=[
            pltpu.VMEM((2, nsub, SUB), jnp.int32),
            pltpu.VMEM((2, CHUNK), jnp.float32),
            pltpu.VMEM((slice_n,), jnp.float32),
            pltpu.VMEM_SHARED((n_pad,), jnp.float32),
            pltpu.SemaphoreType.DMA,
            pltpu.SemaphoreType.DMA,
            pltpu.SemaphoreType.DMA,
            pltpu.SemaphoreType.DMA,
        ],
    )
    def k(dst_hbm, ew_hbm, out_hbm, dstv, eww, zbuf, acc,
          lsem0, lsem1, ssem0, ssem1):
        c = lax.axis_index("c")
        s = lax.axis_index("s")
        wid = c * NS + s
        lsem = (lsem0, lsem1)
        ssem = (ssem0, ssem1)

        @pl.loop(0, slice_n // LANES)
        def _(i):
            zbuf[pl.ds(i * LANES, LANES)] = jnp.zeros((LANES,), jnp.float32)

        pltpu.sync_copy(zbuf, acc.at[pl.ds(s * slice_n, slice_n)])
        plsc.subcore_barrier()

        def load(ch, b):
            base = wid * (n_chunks * CHUNK) + ch * CHUNK
            for j in range(nsub):
                pltpu.async_copy(dst_hbm.at[pl.ds(base + j * SUB, SUB)],
                                 dstv.at[b].at[j], lsem[b])
            pltpu.async_copy(ew_hbm.at[pl.ds(base, CHUNK)], eww.at[b],
                             lsem[b])

        def wait_load(ch, b):
            base = wid * (n_chunks * CHUNK) + ch * CHUNK
            for j in range(nsub):
                pltpu.make_async_copy(dst_hbm.at[pl.ds(base + j * SUB, SUB)],
                                      dstv.at[b].at[j], lsem[b]).wait()
            pltpu.make_async_copy(ew_hbm.at[pl.ds(base, CHUNK)], eww.at[b],
                                  lsem[b]).wait()

        def fire_scat(b):
            for j in range(nsub):
                pltpu.async_copy(eww.at[b].at[pl.ds(j * SUB, SUB)],
                                 acc.at[dstv.at[b].at[j]], ssem[b], add=True)

        def wait_scat(b):
            for j in range(nsub):
                pltpu.make_async_copy(
                    eww.at[b].at[pl.ds(j * SUB, SUB)],
                    acc.at[dstv.at[b].at[j]], ssem[b]).wait()

        load(0, 0)
        load(1, 1)

        @pl.loop(0, n_chunks // 2)
        def _(i):
            c0 = 2 * i

            wait_load(c0, 0)
            fire_scat(0)
            wait_load(c0 + 1, 1)
            fire_scat(1)

            wait_scat(0)

            @pl.when(c0 + 2 < n_chunks)
            def _():
                load(c0 + 2, 0)

            wait_scat(1)

            @pl.when(c0 + 3 < n_chunks)
            def _():
                load(c0 + 3, 1)

        plsc.subcore_barrier()
        pltpu.sync_copy(acc.at[pl.ds(s * slice_n, slice_n)],
                        out_hbm.at[c].at[pl.ds(s * slice_n, slice_n)])

    return k(dst_flat, ew_flat)


def _sc_agg(y, src_flat, dst_flat, ew_flat, h, n_pad, n_chunks):
    slice_n = n_pad // NS
    nsub = CHUNK // SUB
    assert n_chunks % 2 == 0

    @functools.partial(
        pl.kernel,
        out_type=jax.ShapeDtypeStruct((NC, n_pad, h), jnp.float32),
        mesh=_mesh(),
        scratch_types=[
            pltpu.VMEM((2, nsub, SUB), jnp.int32),
            pltpu.VMEM((2, nsub, SUB), jnp.int32),
            pltpu.VMEM((2, CHUNK), jnp.float32),
            pltpu.VMEM((2, CHUNK, h), jnp.float32),
            pltpu.VMEM_SHARED((n_pad, h), jnp.float32),
            pltpu.SemaphoreType.DMA,
            pltpu.SemaphoreType.DMA,
            pltpu.SemaphoreType.DMA,
            pltpu.SemaphoreType.DMA,
        ],
        compiler_params=pltpu.CompilerParams(use_tc_tiling_on_sc=False),
    )
    def k(y_hbm, src_hbm, dst_hbm, ew_hbm, out_hbm,
          srcv, dstv, eww, rows, acc, gsem0, gsem1, ssem0, ssem1):
        c = lax.axis_index("c")
        s = lax.axis_index("s")
        wid = c * NS + s
        gsem = (gsem0, gsem1)
        ssem = (ssem0, ssem1)

        @pl.loop(0, slice_n)
        def _(i):
            for kk in range(h // LANES):
                rows[0, i, pl.ds(kk * LANES, LANES)] = jnp.zeros(
                    (LANES,), jnp.float32)

        pltpu.sync_copy(rows.at[0, pl.ds(0, slice_n)],
                        acc.at[pl.ds(s * slice_n, slice_n)])
        plsc.subcore_barrier()

        def load_idx(ch, b):
            base = wid * (n_chunks * CHUNK) + ch * CHUNK
            for j in range(nsub):
                pltpu.async_copy(src_hbm.at[pl.ds(base + j * SUB, SUB)],
                                 srcv.at[b].at[j], gsem[b])
                pltpu.async_copy(dst_hbm.at[pl.ds(base + j * SUB, SUB)],
                                 dstv.at[b].at[j], gsem[b])
            pltpu.async_copy(ew_hbm.at[pl.ds(base, CHUNK)], eww.at[b],
                             gsem[b])
            for j in range(nsub):
                pltpu.make_async_copy(src_hbm.at[pl.ds(base + j * SUB, SUB)],
                                      srcv.at[b].at[j], gsem[b]).wait()
                pltpu.make_async_copy(dst_hbm.at[pl.ds(base + j * SUB, SUB)],
                                      dstv.at[b].at[j], gsem[b]).wait()
            pltpu.make_async_copy(ew_hbm.at[pl.ds(base, CHUNK)], eww.at[b],
                                  gsem[b]).wait()

        def fire_gather(b):
            for j in range(nsub):
                pltpu.async_copy(y_hbm.at[srcv.at[b].at[j]],
                                 rows.at[b].at[pl.ds(j * SUB, SUB)], gsem[b])

        def wait_gather(b):
            for j in range(nsub):
                pltpu.make_async_copy(
                    y_hbm.at[srcv.at[b].at[j]],
                    rows.at[b].at[pl.ds(j * SUB, SUB)], gsem[b]).wait()

        def fire_scatter(b):
            for j in range(nsub):
                pltpu.async_copy(rows.at[b].at[pl.ds(j * SUB, SUB)],
                                 acc.at[dstv.at[b].at[j]], ssem[b], add=True)

        def wait_scatter(b):
            for j in range(nsub):
                pltpu.make_async_copy(
                    rows.at[b].at[pl.ds(j * SUB, SUB)],
                    acc.at[dstv.at[b].at[j]], ssem[b]).wait()

        def compute(b):
            @pl.loop(0, CHUNK // LANES)
            def _(g):
                wreg = eww.at[b][pl.ds(g * LANES, LANES)]
                for j in range(LANES):
                    e = g * LANES + j
                    wj = wreg[j]
                    for kk in range(h // LANES):
                        sl = pl.ds(kk * LANES, LANES)
                        rows[b, e, sl] = rows[b, e, sl] * wj

        load_idx(0, 0)
        fire_gather(0)
        load_idx(1, 1)
        fire_gather(1)

        @pl.loop(0, n_chunks // 2)
        def _(i):
            c0 = 2 * i

            wait_gather(0)
            compute(0)
            fire_scatter(0)

            wait_gather(1)
            compute(1)
            fire_scatter(1)

            wait_scatter(0)

            @pl.when(c0 + 2 < n_chunks)
            def _():
                load_idx(c0 + 2, 0)
                fire_gather(0)

            wait_scatter(1)

            @pl.when(c0 + 3 < n_chunks)
            def _():
                load_idx(c0 + 3, 1)
                fire_gather(1)
        plsc.subcore_barrier()
        pltpu.sync_copy(acc.at[pl.ds(s * slice_n, slice_n)],
                        out_hbm.at[c].at[pl.ds(s * slice_n, slice_n)])

    return k(y, src_flat, dst_flat, ew_flat)


def _mm_kernel(x_ref, w_ref, o_ref):
    o_ref[...] = jnp.dot(x_ref[...], w_ref[...])


def _tc_matmul(x, w):
    return pl.pallas_call(
        _mm_kernel,
        out_shape=jax.ShapeDtypeStruct((x.shape[0], w.shape[1]), jnp.float32),
    )(x, w)


def _scale_kernel(n, degp_ref, xw_ref, y_ref, dinv_ref):
    deg = degp_ref[0] + degp_ref[1] + 1.0
    dinv = jnp.where(deg > 0, lax.rsqrt(deg), 0.0)
    dinv_col = dinv.reshape(deg.shape[0], 1)[:n]
    y_ref[...] = xw_ref[...] * dinv_col
    dinv_ref[...] = dinv_col


def _tc_scale(degp, xw):
    n = xw.shape[0]
    return pl.pallas_call(
        functools.partial(_scale_kernel, n),
        out_shape=(
            jax.ShapeDtypeStruct(xw.shape, jnp.float32),
            jax.ShapeDtypeStruct((n, 1), jnp.float32),
        ),
    )(degp, xw)


def _mid_kernel(n, dinv_ref, s1_ref, xw_ref, b1_ref, w2_ref,
                hw2_ref, y2_ref):
    dinv = dinv_ref[...]
    s1 = s1_ref[0, :n] + s1_ref[1, :n]
    h = jnp.maximum(dinv * s1 + dinv * dinv * xw_ref[...] + b1_ref[...], 0.0)
    hw2 = jnp.dot(h, w2_ref[...])
    hw2_ref[...] = hw2
    y2_ref[...] = hw2 * dinv


def _tc_mid(dinv_col, s1, xw, b1_row, w2):
    n = xw.shape[0]
    h2 = w2.shape[1]
    return pl.pallas_call(
        functools.partial(_mid_kernel, n),
        out_shape=(
            jax.ShapeDtypeStruct((n, h2), jnp.float32),
            jax.ShapeDtypeStruct((n, h2), jnp.float32),
        ),
    )(dinv_col, s1, xw, b1_row, w2)


def _final_kernel(n, dinv_ref, s2_ref, hw2_ref, b2_ref, o_ref):
    dinv = dinv_ref[...]
    s2 = s2_ref[0, :n] + s2_ref[1, :n]
    o_ref[...] = dinv * s2 + dinv * dinv * hw2_ref[...] + b2_ref[...]


def _tc_final(dinv_col, s2, hw2, b2_row):
    return pl.pallas_call(
        functools.partial(_final_kernel, hw2.shape[0]),
        out_shape=jax.ShapeDtypeStruct(hw2.shape, jnp.float32),
    )(dinv_col, s2, hw2, b2_row)


@jax.jit
def kernel(x, edge_index, edge_weight, W1, b1, W2, b2):
    n, _ = x.shape
    e = edge_weight.shape[0]

    n_chunks = -(-e // (NW * CHUNK))
    e_pad = NW * CHUNK * n_chunks
    pad = e_pad - e
    n_pad = -(-n // (NS * SUB)) * (NS * SUB)

    fill = jnp.arange(pad, dtype=jnp.int32) % n
    src_p = jnp.concatenate([edge_index[0], fill])
    dst_p = jnp.concatenate([edge_index[1], fill])
    ew_p = jnp.concatenate(
        [edge_weight, jnp.zeros((pad,), jnp.float32)])

    xw = _tc_matmul(x, W1)
    degp = _sc_deg(dst_p, ew_p, n_pad, n_chunks)

    y1, dinv_col = _tc_scale(degp, xw)
    s1 = _sc_agg(y1, src_p, dst_p, ew_p, W1.shape[1], n_pad, n_chunks)
    hw2, y2 = _tc_mid(dinv_col, s1, xw, b1.reshape(1, -1), W2)
    s2 = _sc_agg(y2, src_p, dst_p, ew_p, W2.shape[1], n_pad, n_chunks)
    return _tc_final(dinv_col, s2, hw2, b2.reshape(1, -1))
